# baseline (speedup 1.0000x reference)
import functools

import jax
import jax.numpy as jnp
from jax import lax
from jax.experimental import pallas as pl
from jax.experimental.pallas import tpu as pltpu
from jax.experimental.pallas import tpu_sc as plsc

N = 16384
T = 200
B = N * T
D = 32
NC, NS = 2, 16
NW = NC * NS
BPW = B // NW
CN = 1024
NCH = BPW // CN
L = 16

_mesh = plsc.VectorSubcoreMesh(core_axis_name="c", subcore_axis_name="s")


@functools.partial(
    pl.kernel,
    out_type=jax.ShapeDtypeStruct((T, D, N), jnp.float32),
    mesh=_mesh,
    scratch_types=[
        pltpu.VMEM((CN,), jnp.int32),
        pltpu.VMEM((CN, D), jnp.float32),
        pltpu.VMEM((D, CN), jnp.float32),
        pltpu.SemaphoreType.DMA,
    ],
    compiler_params=pltpu.CompilerParams(
        use_tc_tiling_on_sc=False, needs_layout_passes=False
    ),
)
def _gather_t(idx_hbm, table_hbm, out_hbm, idx_v, rows_v, trows_v, sem):
    wid = lax.axis_index("s") * NC + lax.axis_index("c")
    base = wid * BPW
    lane = lax.iota(jnp.int32, L)

    def chunk(k, carry):
        g0 = base + k * CN
        t = g0 >> 14
        n0 = pl.multiple_of(g0 & (N - 1), CN)
        pltpu.sync_copy(idx_hbm.at[pl.ds(g0, CN)], idx_v)
        pltpu.async_copy(table_hbm.at[idx_v], rows_v, sem).wait()

        def jblk(j, c):
            j0 = pl.multiple_of(j * L, L)
            nv = lane + j0
            for f in range(D):
                fv = jnp.full((L,), f, jnp.int32)
                trows_v[f, pl.ds(j0, L)] = plsc.load_gather(rows_v, [nv, fv])
            return c

        lax.fori_loop(0, CN // L, jblk, 0)
        pltpu.sync_copy(trows_v, out_hbm.at[t, :, pl.ds(n0, CN)])
        return carry

    lax.fori_loop(0, NCH, chunk, 0)


def kernel(round_numbers, embedding_table):
    idx_t = round_numbers.T.reshape(-1)
    out_t = _gather_t(idx_t, embedding_table)
    return jnp.transpose(out_t, (2, 0, 1))


# speedup vs baseline: 1.7871x; 1.7871x over previous
import functools

import jax
import jax.numpy as jnp
from jax import lax
from jax.experimental import pallas as pl
from jax.experimental.pallas import tpu as pltpu
from jax.experimental.pallas import tpu_sc as plsc

N = 16384
T = 200
B = N * T
D = 32
NC, NS = 2, 16
NW = NC * NS
BPW = B // NW
CN = 1024
NCH = BPW // CN
L = 16

_mesh = plsc.VectorSubcoreMesh(core_axis_name="c", subcore_axis_name="s")


@functools.partial(
    pl.kernel,
    out_type=jax.ShapeDtypeStruct((T, D // 8, N // 128, 8, 128), jnp.float32),
    mesh=_mesh,
    scratch_types=[
        pltpu.VMEM((CN,), jnp.int32),
        pltpu.VMEM((CN, D), jnp.float32),
        pltpu.VMEM((D // 8, CN // 128, 8, 128), jnp.float32),
        pltpu.SemaphoreType.DMA,
    ],
    compiler_params=pltpu.CompilerParams(
        use_tc_tiling_on_sc=False, needs_layout_passes=False
    ),
)
def _gather_t(idx_hbm, table_hbm, out_hbm, idx_v, rows_v, tile_v, sem):
    wid = lax.axis_index("s") * NC + lax.axis_index("c")
    base = wid * BPW
    lane = lax.iota(jnp.int32, L)
    fvs = [jnp.full((L,), f, jnp.int32) for f in range(D)]

    def chunk(k, carry):
        g0 = base + k * CN
        t = g0 >> 14
        n0 = pl.multiple_of(g0 & (N - 1), CN)
        cb0 = pl.multiple_of(n0 >> 7, CN // 128)
        pltpu.sync_copy(idx_hbm.at[pl.ds(g0, CN)], idx_v)
        pltpu.async_copy(table_hbm.at[idx_v], rows_v, sem).wait()

        def jblk(j, c):
            j0 = pl.multiple_of(j * L, L)
            nv = lane + j0
            vals = [
                plsc.load_gather(rows_v, [nv, fvs[f]]) for f in range(D)
            ]
            m = j // 8
            jt = pl.multiple_of((j0 % 128), L)
            for f in range(D):
                tile_v[f // 8, m, f % 8, pl.ds(jt, L)] = vals[f]
            return c

        lax.fori_loop(0, CN // L, jblk, 0)
        pltpu.sync_copy(tile_v, out_hbm.at[t, :, pl.ds(cb0, CN // 128)])
        return carry

    lax.fori_loop(0, NCH, chunk, 0)


def kernel(round_numbers, embedding_table):
    idx_t = round_numbers.T.reshape(-1)
    out5 = _gather_t(idx_t, embedding_table)
    out_t = out5.transpose(0, 1, 3, 2, 4).reshape(T, D, N)
    return jnp.transpose(out_t, (2, 0, 1))


# speedup vs baseline: 2.2791x; 1.2753x over previous
import functools

import jax
import jax.numpy as jnp
from jax import lax
from jax.experimental import pallas as pl
from jax.experimental.pallas import tpu as pltpu
from jax.experimental.pallas import tpu_sc as plsc

N = 16384
T = 200
B = N * T
D = 32
NC, NS = 2, 16
NW = NC * NS
BPW = B // NW
CN = 512
NCH = BPW // CN
NGRP = NCH // 2
CB = CN // 128
L = 16

_mesh = plsc.VectorSubcoreMesh(core_axis_name="c", subcore_axis_name="s")


@functools.partial(
    pl.kernel,
    out_type=jax.ShapeDtypeStruct((T, D // 8, N // 128, 8, 128), jnp.float32),
    mesh=_mesh,
    scratch_types=[
        pltpu.VMEM((2, CN), jnp.int32),
        pltpu.VMEM((2, CN, D), jnp.float32),
        pltpu.VMEM((2, D // 8, CB, 8, 128), jnp.float32),
        [pltpu.SemaphoreType.DMA] * 2,
        [pltpu.SemaphoreType.DMA] * 2,
        [pltpu.SemaphoreType.DMA] * 2,
    ],
    compiler_params=pltpu.CompilerParams(
        use_tc_tiling_on_sc=False, needs_layout_passes=False
    ),
)
def _gather_t(idx_hbm, table_hbm, out_hbm, idx_v, rows_v, tile_v,
              isems, gsems, wsems):
    wid = lax.axis_index("s") * NC + lax.axis_index("c")
    base = wid * BPW
    lane = lax.iota(jnp.int32, L)
    fvs = [jnp.full((L,), f, jnp.int32) for f in range(D)]

    def coords(k):
        g0 = base + k * CN
        t = g0 >> 14
        cb0 = pl.multiple_of((g0 & (N - 1)) >> 7, CB)
        return g0, t, cb0

    def transpose_chunk(b):
        def jblk(j, c):
            j0 = pl.multiple_of(j * L, L)
            nv = lane + j0
            vals = [
                plsc.load_gather(rows_v.at[b], [nv, fvs[f]]) for f in range(D)
            ]
            m = j // 8
            jt = pl.multiple_of(j0 % 128, L)
            for f in range(D):
                tile_v[b, f // 8, m, f % 8, pl.ds(jt, L)] = vals[f]
            return c

        lax.fori_loop(0, CN // L, jblk, 0)

    for b in range(2):
        g0, _, _ = coords(b)
        pltpu.sync_copy(idx_hbm.at[pl.ds(g0, CN)], idx_v.at[b])
        pltpu.async_copy(table_hbm.at[idx_v.at[b]], rows_v.at[b], gsems[b])

    def group(g, carry):
        for b in range(2):
            k = 2 * g + b
            g0, t, cb0 = coords(k)
            pltpu.make_async_copy(
                table_hbm.at[idx_v.at[b]], rows_v.at[b], gsems[b]
            ).wait()

            @pl.when(g < NGRP - 1)
            def _pfi(b=b, k=k):
                g0n, _, _ = coords(k + 2)
                pltpu.async_copy(
                    idx_hbm.at[pl.ds(g0n, CN)], idx_v.at[b], isems[b]
                )

            @pl.when(g > 0)
            def _drain(b=b, k=k):
                _, tp, cb0p = coords(k - 2)
                pltpu.make_async_copy(
                    tile_v.at[b], out_hbm.at[tp, :, pl.ds(cb0p, CB)], wsems[b]
                ).wait()

            transpose_chunk(b)
            pltpu.async_copy(
                tile_v.at[b], out_hbm.at[t, :, pl.ds(cb0, CB)], wsems[b]
            )

            @pl.when(g < NGRP - 1)
            def _pfg(b=b):
                pltpu.make_async_copy(
                    idx_hbm.at[pl.ds(0, CN)], idx_v.at[b], isems[b]
                ).wait()
                pltpu.async_copy(
                    table_hbm.at[idx_v.at[b]], rows_v.at[b], gsems[b]
                )

        return carry

    lax.fori_loop(0, NGRP, group, 0)
    for b in range(2):
        k = NCH - 2 + b
        _, t, cb0 = coords(k)
        pltpu.make_async_copy(
            tile_v.at[b], out_hbm.at[t, :, pl.ds(cb0, CB)], wsems[b]
        ).wait()


def kernel(round_numbers, embedding_table):
    idx_t = round_numbers.T.reshape(-1)
    out5 = _gather_t(idx_t, embedding_table)
    out_t = out5.transpose(0, 1, 3, 2, 4).reshape(T, D, N)
    return jnp.transpose(out_t, (2, 0, 1))
